# baseline (speedup 1.0000x reference)
import functools

import jax
import jax.numpy as jnp
from jax import lax
from jax.experimental import pallas as pl
from jax.experimental.pallas import tpu as pltpu
from jax.experimental.pallas import tpu_sc as plsc

_N = 10000
_C = 128
_NOPS = 3
_NCORES = 2
_NSUB = 16
_NW = _NCORES * _NSUB
_CHUNK = 64
_NBUF = 4
_SB = 32
_NP = 10240
_RPT = _NP // _NSUB
_LANE = 16

_ADMM_ITER = 10
_RHO = 1.05
_MU2_MAX = 1000000.0
_NU = (0.0, 1.0, 0.25)


def _sc_spmm_kernel(nch, per_j_drain):
    assert nch % _SB == 0
    nsb = nch // _SB
    if per_j_drain:
        out_type = jax.ShapeDtypeStruct((_NCORES, _NOPS, _NP, _C), jnp.float32)
    else:
        out_type = jax.ShapeDtypeStruct((_NCORES, _NP, _C), jnp.float32)
    mesh = plsc.VectorSubcoreMesh(core_axis_name="c", subcore_axis_name="s")

    @functools.partial(
        pl.kernel,
        out_type=out_type,
        mesh=mesh,
        scratch_types=[
            pltpu.VMEM((_SB, _CHUNK), jnp.int32),
            pltpu.VMEM((_SB, _CHUNK), jnp.int32),
            pltpu.VMEM((_SB, _CHUNK), jnp.float32),
            pltpu.VMEM((_NBUF, _CHUNK, _C), jnp.float32),
            pltpu.VMEM_SHARED((_NP, _C), jnp.float32),
            pltpu.SemaphoreType.DMA((_NBUF,)),
            pltpu.SemaphoreType.DMA((_NBUF,)),
        ],
    )
    def body(src, cols, rows, vals, out, cols_v, rows_v, vals_v, gbuf, acc,
             gsem, ssem):
        c = lax.axis_index("c")
        s = lax.axis_index("s")
        w = c * _NSUB + s

        def zero_acc():
            def zrow(r, carry):
                for q in range(_C // _LANE):
                    gbuf[0, r, pl.ds(q * _LANE, _LANE)] = jnp.zeros(
                        (_LANE,), jnp.float32)
                return carry
            lax.fori_loop(0, _CHUNK, zrow, 0)
            for k in range(_RPT // _CHUNK):
                pltpu.sync_copy(
                    gbuf.at[0],
                    acc.at[pl.ds(s * _RPT + k * _CHUNK, _CHUNK)])

        def fire_gather(ch, b):
            pltpu.async_copy(src.at[cols_v.at[ch]], gbuf.at[b], gsem.at[b])

        def wait_gather(ch, b):
            pltpu.make_async_copy(src.at[cols_v.at[ch]], gbuf.at[b],
                                  gsem.at[b]).wait()

        def scale_rows(ch, b):
            def gbody(g, carry):
                v16 = vals_v[ch, pl.ds(g * _LANE, _LANE)]
                for i in range(_LANE):
                    r = g * _LANE + i
                    s = v16[i]
                    for q in range(_C // _LANE):
                        sl = pl.ds(q * _LANE, _LANE)
                        gbuf[b, r, sl] = gbuf[b, r, sl] * s
                return carry
            lax.fori_loop(0, _CHUNK // _LANE, gbody, 0)

        def scatter_add(ch, b):
            pltpu.sync_copy(gbuf.at[b], acc.at[rows_v.at[ch]], add=True)

        def step(ch, b):
            @pl.when(ch + _NBUF - 1 < _SB)
            def _():
                fire_gather(ch + _NBUF - 1, (b + _NBUF - 1) % _NBUF)

            wait_gather(ch, b)
            scale_rows(ch, b)
            scatter_add(ch, b)

        def run_subslab(j, sb):
            base = sb * _SB
            pltpu.sync_copy(cols.at[w, j, pl.ds(base, _SB)], cols_v)
            pltpu.sync_copy(rows.at[w, j, pl.ds(base, _SB)], rows_v)
            pltpu.sync_copy(vals.at[w, j, pl.ds(base, _SB)], vals_v)
            for p in range(_NBUF - 1):
                fire_gather(p, p)

            def iter4(it, carry):
                for u in range(_NBUF):
                    step(it * _NBUF + u, u)
                return carry
            lax.fori_loop(0, _SB // _NBUF, iter4, 0)

        def run_j(j):
            def sbody(sb, carry):
                run_subslab(j, sb)
                return carry
            lax.fori_loop(0, nsb, sbody, 0)

        def drain(out_slice):
            pltpu.sync_copy(acc.at[pl.ds(s * _RPT, _RPT)], out_slice)

        if per_j_drain:
            def jbody(j, carry):
                zero_acc()
                plsc.subcore_barrier()
                run_j(j)
                plsc.subcore_barrier()
                drain(out.at[c, j, pl.ds(s * _RPT, _RPT)])
                plsc.subcore_barrier()
                return carry
            lax.fori_loop(0, _NOPS, jbody, 0)
        else:
            zero_acc()
            plsc.subcore_barrier()

            def jbody(j, carry):
                run_j(j)
                return carry
            lax.fori_loop(0, _NOPS, jbody, 0)
            plsc.subcore_barrier()
            drain(out.at[c, pl.ds(s * _RPT, _RPT)])

    return body


def _uk0_call(F, dcol, mu2):
    rb = 400
    grid = (_N // rb,)

    def body(f_ref, d_ref, o_ref):
        dd = d_ref[...]
        o_ref[...] = (dd * f_ref[...]) / (dd + mu2)

    return pl.pallas_call(
        body,
        grid=grid,
        in_specs=[pl.BlockSpec((rb, _C), lambda i: (i, 0)),
                  pl.BlockSpec((rb, 1), lambda i: (i, 0))],
        out_specs=pl.BlockSpec((rb, _C), lambda i: (i, 0)),
        out_shape=jax.ShapeDtypeStruct((_N, _C), jnp.float32),
    )(F, dcol)


def _uk_call(F, dcol, Sp, mu2):
    rb = 400
    grid = (_N // rb,)

    def body(f_ref, d_ref, sp_ref, o_ref):
        dd = d_ref[...]
        s = sp_ref[0] + sp_ref[1]
        o_ref[...] = (dd * f_ref[...] + s) / (dd + mu2)

    return pl.pallas_call(
        body,
        grid=grid,
        in_specs=[pl.BlockSpec((rb, _C), lambda i: (i, 0)),
                  pl.BlockSpec((rb, 1), lambda i: (i, 0)),
                  pl.BlockSpec((2, rb, _C), lambda i: (0, i, 0))],
        out_specs=pl.BlockSpec((rb, _C), lambda i: (i, 0)),
        out_shape=jax.ShapeDtypeStruct((_N, _C), jnp.float32),
    )(F, dcol, Sp)


def _update_call(WUp, Lam, nud, mu2, mu2n):
    rb = 400
    grid = (_NOPS, _N // rb)
    inv = 1.0 / mu2

    def body(wu_ref, lam_ref, nud_ref, lam_out, a_out):
        wu = wu_ref[0, 0] + wu_ref[1, 0]
        lam = lam_ref[0]
        eta = nud_ref[0] * inv
        qx = wu - lam * inv
        q = jnp.maximum(qx - eta, 0.0) - jnp.maximum(-qx - eta, 0.0)
        lnew = lam + mu2 * (q - wu)
        lam_out[0] = lnew
        a_out[0] = mu2n * q + lnew

    return pl.pallas_call(
        body,
        grid=grid,
        in_specs=[
            pl.BlockSpec((2, 1, rb, _C), lambda j, i: (0, j, i, 0)),
            pl.BlockSpec((1, rb, _C), lambda j, i: (j, i, 0)),
            pl.BlockSpec((1, rb, 1), lambda j, i: (j, i, 0)),
        ],
        out_specs=[
            pl.BlockSpec((1, rb, _C), lambda j, i: (j, i, 0)),
            pl.BlockSpec((1, rb, _C), lambda j, i: (j, i, 0)),
        ],
        out_shape=[jax.ShapeDtypeStruct((_NOPS, _N, _C), jnp.float32),
                   jax.ShapeDtypeStruct((_NOPS, _N, _C), jnp.float32)],
    )(WUp, Lam, nud)


def _prep_edges(idx_list, val_list):
    e = val_list[0].shape[0]
    ejt = -(-e // _NW)
    nch = -(-ejt // _CHUNK)
    nch = -(-nch // _SB) * _SB
    padj = _NW * nch * _CHUNK
    rows_j, cols_a, cols_b, vals_j = [], [], [], []
    for j, (idx, val) in enumerate(zip(idx_list, val_list)):
        rows = idx[0].astype(jnp.int32)
        cols = idx[1].astype(jnp.int32)
        pad = padj - e
        rows = jnp.pad(rows, (0, pad))
        cols = jnp.pad(cols, (0, pad))
        val = jnp.pad(val, (0, pad))
        rows_j.append(rows)
        cols_b.append(cols)
        cols_a.append(cols + j * _N)
        vals_j.append(val)

    def pack(xs):
        st = jnp.stack(xs)
        return st.reshape(_NOPS, _NW, nch, _CHUNK).transpose(1, 0, 2, 3)

    return nch, pack(rows_j), pack(cols_a), pack(cols_b), pack(vals_j)


def kernel(F, d, W0_idx, W0_val, W1_idx, W1_val, W2_idx, W2_val):
    nch, rows_t, colsa_t, colsb_t, vals_t = _prep_edges(
        [W0_idx, W1_idx, W2_idx], [W0_val, W1_val, W2_val])

    stage_a = _sc_spmm_kernel(nch, per_j_drain=False)
    stage_b = _sc_spmm_kernel(nch, per_j_drain=True)

    mu2s = [1.0]
    for _ in range(_ADMM_ITER):
        mu2s.append(min(_RHO * mu2s[-1], _MU2_MAX))

    dcol = d[:, None]
    nud = jnp.stack([nu * d for nu in _NU])[:, :, None]

    uk = _uk0_call(F, dcol, mu2s[0])
    lam = jnp.zeros((_NOPS, _N, _C), jnp.float32)
    for k in range(_ADMM_ITER - 1):
        mu2, mu2n = mu2s[k], mu2s[k + 1]
        wup = stage_b(uk, colsb_t, rows_t, vals_t)
        lam, a = _update_call(wup, lam, nud, mu2, mu2n)
        sp = stage_a(a.reshape(_NOPS * _N, _C), colsa_t, rows_t, vals_t)
        uk = _uk_call(F, dcol, sp, mu2n)
    return uk


# speedup vs baseline: 1.0177x; 1.0177x over previous
import functools

import jax
import jax.numpy as jnp
from jax import lax
from jax.experimental import pallas as pl
from jax.experimental.pallas import tpu as pltpu
from jax.experimental.pallas import tpu_sc as plsc

_N = 10000
_C = 128
_NOPS = 3
_NCORES = 2
_NSUB = 16
_NW = _NCORES * _NSUB
_CHUNK = 64
_NBUF = 4
_SB = 32
_NP = 10240
_RPT = _NP // _NSUB
_LANE = 16

_ADMM_ITER = 10
_RHO = 1.05
_MU2_MAX = 1000000.0
_NU = (0.0, 1.0, 0.25)


def _sc_spmm_kernel(nch, per_j_drain):
    assert nch % _SB == 0
    nsb = nch // _SB
    if per_j_drain:
        out_type = jax.ShapeDtypeStruct((_NCORES, _NOPS, _NP, _C), jnp.float32)
    else:
        out_type = jax.ShapeDtypeStruct((_NCORES, _NP, _C), jnp.float32)
    mesh = plsc.VectorSubcoreMesh(core_axis_name="c", subcore_axis_name="s")

    @functools.partial(
        pl.kernel,
        out_type=out_type,
        mesh=mesh,
        scratch_types=[
            pltpu.VMEM((_SB, _CHUNK), jnp.int32),
            pltpu.VMEM((_SB, _CHUNK), jnp.int32),
            pltpu.VMEM((_SB, _CHUNK), jnp.float32),
            pltpu.VMEM((_NBUF, _CHUNK, _C), jnp.float32),
            pltpu.VMEM_SHARED((_NP, _C), jnp.float32),
            pltpu.SemaphoreType.DMA((_NBUF,)),
            pltpu.SemaphoreType.DMA((_NBUF,)),
        ],
    )
    def body(src, cols, rows, vals, out, cols_v, rows_v, vals_v, gbuf, acc,
             gsem, ssem):
        c = lax.axis_index("c")
        s = lax.axis_index("s")
        w = c * _NSUB + s

        def zero_acc():
            def zrow(r, carry):
                for q in range(_C // _LANE):
                    gbuf[0, r, pl.ds(q * _LANE, _LANE)] = jnp.zeros(
                        (_LANE,), jnp.float32)
                return carry
            lax.fori_loop(0, _CHUNK, zrow, 0)
            for k in range(_RPT // _CHUNK):
                pltpu.sync_copy(
                    gbuf.at[0],
                    acc.at[pl.ds(s * _RPT + k * _CHUNK, _CHUNK)])

        def fire_gather(ch, b):
            pltpu.async_copy(src.at[cols_v.at[ch]], gbuf.at[b], gsem.at[b])

        def wait_gather(ch, b):
            pltpu.make_async_copy(src.at[cols_v.at[ch]], gbuf.at[b],
                                  gsem.at[b]).wait()

        def scale_rows(ch, b):
            def gbody(g, carry):
                v16 = vals_v[ch, pl.ds(g * _LANE, _LANE)]
                for i in range(_LANE):
                    r = g * _LANE + i
                    s = v16[i]
                    for q in range(_C // _LANE):
                        sl = pl.ds(q * _LANE, _LANE)
                        gbuf[b, r, sl] = gbuf[b, r, sl] * s
                return carry
            lax.fori_loop(0, _CHUNK // _LANE, gbody, 0)

        def fire_scatter(ch, b):
            pltpu.async_copy(gbuf.at[b], acc.at[rows_v.at[ch]], ssem.at[b],
                             add=True)

        def wait_scatter(ch, b):
            pltpu.make_async_copy(gbuf.at[b], acc.at[rows_v.at[ch]],
                                  ssem.at[b]).wait()

        def step(ch, b):
            @pl.when(ch + _NBUF - 1 < _SB)
            def _():
                nb = (b + _NBUF - 1) % _NBUF

                @pl.when(ch >= 1)
                def _():
                    wait_scatter(ch - 1, nb)
                fire_gather(ch + _NBUF - 1, nb)

            wait_gather(ch, b)
            scale_rows(ch, b)
            fire_scatter(ch, b)

        def run_subslab(j, sb):
            base = sb * _SB
            pltpu.sync_copy(cols.at[w, j, pl.ds(base, _SB)], cols_v)
            pltpu.sync_copy(rows.at[w, j, pl.ds(base, _SB)], rows_v)
            pltpu.sync_copy(vals.at[w, j, pl.ds(base, _SB)], vals_v)
            for p in range(_NBUF - 1):
                fire_gather(p, p)

            def iter4(it, carry):
                for u in range(_NBUF):
                    step(it * _NBUF + u, u)
                return carry
            lax.fori_loop(0, _SB // _NBUF, iter4, 0)
            for u in range(_NBUF):
                wait_scatter(_SB - _NBUF + u, u)

        def run_j(j):
            def sbody(sb, carry):
                run_subslab(j, sb)
                return carry
            lax.fori_loop(0, nsb, sbody, 0)

        def drain(out_slice):
            pltpu.sync_copy(acc.at[pl.ds(s * _RPT, _RPT)], out_slice)

        if per_j_drain:
            def jbody(j, carry):
                zero_acc()
                plsc.subcore_barrier()
                run_j(j)
                plsc.subcore_barrier()
                drain(out.at[c, j, pl.ds(s * _RPT, _RPT)])
                plsc.subcore_barrier()
                return carry
            lax.fori_loop(0, _NOPS, jbody, 0)
        else:
            zero_acc()
            plsc.subcore_barrier()

            def jbody(j, carry):
                run_j(j)
                return carry
            lax.fori_loop(0, _NOPS, jbody, 0)
            plsc.subcore_barrier()
            drain(out.at[c, pl.ds(s * _RPT, _RPT)])

    return body


def _uk0_call(F, dcol, mu2):
    rb = 400
    grid = (_N // rb,)

    def body(f_ref, d_ref, o_ref):
        dd = d_ref[...]
        o_ref[...] = (dd * f_ref[...]) / (dd + mu2)

    return pl.pallas_call(
        body,
        grid=grid,
        in_specs=[pl.BlockSpec((rb, _C), lambda i: (i, 0)),
                  pl.BlockSpec((rb, 1), lambda i: (i, 0))],
        out_specs=pl.BlockSpec((rb, _C), lambda i: (i, 0)),
        out_shape=jax.ShapeDtypeStruct((_N, _C), jnp.float32),
    )(F, dcol)


def _uk_call(F, dcol, Sp, mu2):
    rb = 400
    grid = (_N // rb,)

    def body(f_ref, d_ref, sp_ref, o_ref):
        dd = d_ref[...]
        s = sp_ref[0] + sp_ref[1]
        o_ref[...] = (dd * f_ref[...] + s) / (dd + mu2)

    return pl.pallas_call(
        body,
        grid=grid,
        in_specs=[pl.BlockSpec((rb, _C), lambda i: (i, 0)),
                  pl.BlockSpec((rb, 1), lambda i: (i, 0)),
                  pl.BlockSpec((2, rb, _C), lambda i: (0, i, 0))],
        out_specs=pl.BlockSpec((rb, _C), lambda i: (i, 0)),
        out_shape=jax.ShapeDtypeStruct((_N, _C), jnp.float32),
    )(F, dcol, Sp)


def _update_call(WUp, Lam, nud, mu2, mu2n):
    rb = 400
    grid = (_NOPS, _N // rb)
    inv = 1.0 / mu2

    def body(wu_ref, lam_ref, nud_ref, lam_out, a_out):
        wu = wu_ref[0, 0] + wu_ref[1, 0]
        lam = lam_ref[0]
        eta = nud_ref[0] * inv
        qx = wu - lam * inv
        q = jnp.maximum(qx - eta, 0.0) - jnp.maximum(-qx - eta, 0.0)
        lnew = lam + mu2 * (q - wu)
        lam_out[0] = lnew
        a_out[0] = mu2n * q + lnew

    return pl.pallas_call(
        body,
        grid=grid,
        in_specs=[
            pl.BlockSpec((2, 1, rb, _C), lambda j, i: (0, j, i, 0)),
            pl.BlockSpec((1, rb, _C), lambda j, i: (j, i, 0)),
            pl.BlockSpec((1, rb, 1), lambda j, i: (j, i, 0)),
        ],
        out_specs=[
            pl.BlockSpec((1, rb, _C), lambda j, i: (j, i, 0)),
            pl.BlockSpec((1, rb, _C), lambda j, i: (j, i, 0)),
        ],
        out_shape=[jax.ShapeDtypeStruct((_NOPS, _N, _C), jnp.float32),
                   jax.ShapeDtypeStruct((_NOPS, _N, _C), jnp.float32)],
    )(WUp, Lam, nud)


def _prep_edges(idx_list, val_list):
    e = val_list[0].shape[0]
    ejt = -(-e // _NW)
    nch = -(-ejt // _CHUNK)
    nch = -(-nch // _SB) * _SB
    padj = _NW * nch * _CHUNK
    rows_j, cols_a, cols_b, vals_j = [], [], [], []
    for j, (idx, val) in enumerate(zip(idx_list, val_list)):
        rows = idx[0].astype(jnp.int32)
        cols = idx[1].astype(jnp.int32)
        pad = padj - e
        rows = jnp.pad(rows, (0, pad))
        cols = jnp.pad(cols, (0, pad))
        val = jnp.pad(val, (0, pad))
        rows_j.append(rows)
        cols_b.append(cols)
        cols_a.append(cols + j * _N)
        vals_j.append(val)

    def pack(xs):
        st = jnp.stack(xs)
        return st.reshape(_NOPS, _NW, nch, _CHUNK).transpose(1, 0, 2, 3)

    return nch, pack(rows_j), pack(cols_a), pack(cols_b), pack(vals_j)


def kernel(F, d, W0_idx, W0_val, W1_idx, W1_val, W2_idx, W2_val):
    nch, rows_t, colsa_t, colsb_t, vals_t = _prep_edges(
        [W0_idx, W1_idx, W2_idx], [W0_val, W1_val, W2_val])

    stage_a = _sc_spmm_kernel(nch, per_j_drain=False)
    stage_b = _sc_spmm_kernel(nch, per_j_drain=True)

    mu2s = [1.0]
    for _ in range(_ADMM_ITER):
        mu2s.append(min(_RHO * mu2s[-1], _MU2_MAX))

    dcol = d[:, None]
    nud = jnp.stack([nu * d for nu in _NU])[:, :, None]

    uk = _uk0_call(F, dcol, mu2s[0])
    lam = jnp.zeros((_NOPS, _N, _C), jnp.float32)
    for k in range(_ADMM_ITER - 1):
        mu2, mu2n = mu2s[k], mu2s[k + 1]
        wup = stage_b(uk, colsb_t, rows_t, vals_t)
        lam, a = _update_call(wup, lam, nud, mu2, mu2n)
        sp = stage_a(a.reshape(_NOPS * _N, _C), colsa_t, rows_t, vals_t)
        uk = _uk_call(F, dcol, sp, mu2n)
    return uk
